# baseline (speedup 1.0000x reference)
import functools
import jax
import jax.numpy as jnp
from jax.experimental import pallas as pl
from jax.experimental.pallas import tpu as pltpu
from jax.experimental.pallas import tpu_sc as plsc


def _f32(x):
    return jnp.float32(x)


def _erfc_cephes(x):
    abs_x = jnp.abs(x)
    xx = x * x
    ep = _f32(7.853861353153693e-5)
    for c in (-8.010193625184903e-4, 5.188327685732524e-3,
              -2.685381193529856e-2, 1.128358514861418e-1,
              -3.761262582423300e-1, 1.128379165726710e+0):
        ep = ep * xx + _f32(c)
    branch_lt1 = _f32(1.0) - x * ep
    nxx = -xx
    z = jnp.exp(nxx)
    q = _f32(1.0) / abs_x
    zq = z * q
    w = _f32(1.0) / xx
    pp = _f32(2.326819970068386e-2)
    for c in (-1.387039388740657e-1, 3.687424674597105e-1,
              -5.824733027278666e-1, 6.210004621745983e-1,
              -4.944515323274145e-1, 3.404879937665872e-1,
              -2.741127028184656e-1, 5.638259427386472e-1):
        pp = pp * w + _f32(c)
    rr = _f32(-1.047766399936249e+1)
    for c in (1.297719955372516e+1, -7.495518717768503e+0,
              2.921019019210786e+0, -1.015265279202700e+0,
              4.218463358204948e-1, -2.820767439740514e-1,
              5.641895067754075e-1):
        rr = rr * w + _f32(c)
    p = jnp.where(abs_x < _f32(2.0), pp, rr)
    y = zq * p
    y = jnp.where(nxx < _f32(-88.72283905206835), _f32(0.0), y)
    res_big = jnp.where(x < _f32(0.0), _f32(2.0) - y, y)
    return jnp.where(abs_x < _f32(1.0), branch_lt1, res_big)


_SQRT_HALF = 0.5 ** 0.5


def _gelu(x):
    return 0.5 * x * _erfc_cephes(-x * _f32(_SQRT_HALF))


def _gelu_fast(x):
    return 0.5 * x * (1.0 + jax.lax.erf(x * _f32(_SQRT_HALF)))


def _fused_body(f_ref, w1_ref, b1_ref, w2_ref, b2_ref, pw1_ref, pb1_ref,
                pw2_ref, pb2_ref, sidx_ref, p_ref, scores_scr):
    g = pl.program_id(0)
    T = _SORT_WAYS
    n_mlp = 32

    @pl.when(g < n_mlp)
    def _mlp_step():
        b = g // 8
        n = g % 8
        f = f_ref[0]
        hid = _gelu(jnp.dot(f, w1_ref[...],
                            preferred_element_type=jnp.float32) + b1_ref[...])
        sc = jnp.dot(hid, w2_ref[...],
                     preferred_element_type=jnp.float32) + b2_ref[...]
        sct = jnp.transpose(sc, (1, 0))[:3]
        sct3 = sct.reshape(3, sct.shape[1] // _C, _C)
        scores_scr[pl.ds(b * 3, 3), pl.ds(n * 16, 16), :] = sct3
        ph = _gelu_fast(jnp.dot(f, pw1_ref[...],
                                preferred_element_type=jnp.float32)
                        + pb1_ref[...])
        p_ref[0] = jnp.dot(ph, pw2_ref[...],
                           preferred_element_type=jnp.float32) + pb2_ref[...]

    @pl.when(g >= n_mlp)
    def _sort_step():
        pid = g - n_mlp
        r_iota = jax.lax.broadcasted_iota(jnp.int32, (_R, _C), 0)
        c_iota = jax.lax.broadcasted_iota(jnp.int32, (_R, _C), 1)
        pos = r_iota * _C + c_iota
        n = _R * _C
        ss = [scores_scr[pid * T + t] for t in range(T)]
        ii = [pos for _ in range(T)]
        k = 2
        while k <= n:
            j = k // 2
            while j >= 1:
                is_lower = (pos & j) == 0
                block_fwd = (pos & k) == 0
                fwd_dir = block_fwd == is_lower
                for t in range(T):
                    ps = _partner(ss[t], j)
                    pi = _partner(ii[t], j)
                    self_first = (ss[t] > ps) | ((ss[t] == ps) & (ii[t] < pi))
                    keep_self = self_first == fwd_dir
                    ss[t] = jnp.where(keep_self, ss[t], ps)
                    ii[t] = jnp.where(keep_self, ii[t], pi)
                j //= 2
            k *= 2
        for t in range(T):
            base = ((pid * T + t) // 3) * n
            sidx_ref[t] = ii[t] + base


def _scores_and_prompt(features, W1cat, b1cat, W2blk, b2cat,
                       prompt_W1, prompt_b1, prompt_W2, prompt_b2, TN=2048):
    B, N, h = features.shape
    n_mlp = B * (N // TN)
    n_sort = 3 * B // _SORT_WAYS
    T = _SORT_WAYS

    def f_map(g):
        gm = jnp.minimum(g, n_mlp - 1)
        return (gm // 8, gm % 8, 0)

    return pl.pallas_call(
        _fused_body,
        grid=(n_mlp + n_sort,),
        in_specs=[
            pl.BlockSpec((1, TN, h), f_map),
            pl.BlockSpec((h, 384), lambda g: (0, 0)),
            pl.BlockSpec((384,), lambda g: (0,)),
            pl.BlockSpec((384, 8), lambda g: (0, 0)),
            pl.BlockSpec((8,), lambda g: (0,)),
            pl.BlockSpec((h, h), lambda g: (0, 0)),
            pl.BlockSpec((h,), lambda g: (0,)),
            pl.BlockSpec((h, h), lambda g: (0, 0)),
            pl.BlockSpec((h,), lambda g: (0,)),
        ],
        out_specs=[
            pl.BlockSpec((T, _R, _C),
                         lambda g: (jnp.maximum(g - n_mlp, 0), 0, 0)),
            pl.BlockSpec((1, TN, h), f_map),
        ],
        out_shape=[
            jax.ShapeDtypeStruct((3 * B, _R, _C), jnp.int32),
            jax.ShapeDtypeStruct((B, N, h), jnp.float32),
        ],
        scratch_shapes=[pltpu.VMEM((3 * B, _R, _C), jnp.float32)],
    )(features, W1cat, b1cat, W2blk, b2cat,
      prompt_W1, prompt_b1, prompt_W2, prompt_b2)


_R, _C = 128, 128
_SORT_WAYS = 2


def _partner(x, j):
    if j < _C:
        fwd = pltpu.roll(x, _C - j, axis=1)
        bwd = pltpu.roll(x, j, axis=1)
        bit = (jax.lax.broadcasted_iota(jnp.int32, (_R, _C), 1) & j) == 0
    else:
        m = j // _C
        fwd = pltpu.roll(x, _R - m, axis=0)
        bwd = pltpu.roll(x, m, axis=0)
        bit = (jax.lax.broadcasted_iota(jnp.int32, (_R, _C), 0) & m) == 0
    return jnp.where(bit, fwd, bwd)


def _sort_body(s_ref, idx_ref):
    T = _SORT_WAYS
    r_iota = jax.lax.broadcasted_iota(jnp.int32, (_R, _C), 0)
    c_iota = jax.lax.broadcasted_iota(jnp.int32, (_R, _C), 1)
    pos = r_iota * _C + c_iota
    ss = [s_ref[t] for t in range(T)]
    ii = [pos for _ in range(T)]
    n = _R * _C
    k = 2
    while k <= n:
        j = k // 2
        while j >= 1:
            is_lower = (pos & j) == 0
            block_fwd = (pos & k) == 0
            fwd_dir = block_fwd == is_lower
            for t in range(T):
                ps = _partner(ss[t], j)
                pi = _partner(ii[t], j)
                self_first = (ss[t] > ps) | ((ss[t] == ps) & (ii[t] < pi))
                keep_self = self_first == fwd_dir
                ss[t] = jnp.where(keep_self, ss[t], ps)
                ii[t] = jnp.where(keep_self, ii[t], pi)
            j //= 2
        k *= 2
    for t in range(T):
        base = ((pl.program_id(0) * T + t) // 3) * n
        idx_ref[t] = ii[t] + base


def _argsort_desc(scores):
    G = scores.shape[0]
    T = _SORT_WAYS
    return pl.pallas_call(
        _sort_body,
        grid=(G // T,),
        in_specs=[pl.BlockSpec((T, _R, _C), lambda g: (g, 0, 0))],
        out_specs=pl.BlockSpec((T, _R, _C), lambda g: (g, 0, 0)),
        out_shape=jax.ShapeDtypeStruct((G, _R, _C), jnp.int32),
    )(scores)


_SC_CH = 224


def _sc_gather(table, idx_flat):
    M = idx_flat.shape[0]
    D = table.shape[1]
    info = plsc.get_sparse_core_info()
    NC, NS = info.num_cores, info.num_subcores
    NW = NC * NS
    b_per_w = M // NW
    steps = b_per_w // _SC_CH
    mesh = plsc.VectorSubcoreMesh(core_axis_name="c", subcore_axis_name="s")

    @functools.partial(
        pl.kernel, mesh=mesh,
        out_type=jax.ShapeDtypeStruct((M, D), jnp.float32),
        scratch_types=[
            pltpu.VMEM((b_per_w,), jnp.int32),
            pltpu.VMEM((_SC_CH, D), jnp.float32),
            pltpu.VMEM((_SC_CH, D), jnp.float32),
            pltpu.SemaphoreType.DMA,
            pltpu.SemaphoreType.DMA,
            pltpu.SemaphoreType.DMA,
            pltpu.SemaphoreType.DMA,
        ],
    )
    def gath(table_hbm, idx_hbm, out_hbm, idx_v, rows0, rows1,
             semg0, semg1, semo0, semo1):
        wid = jax.lax.axis_index("s") * NC + jax.lax.axis_index("c")
        base = wid * b_per_w
        pltpu.sync_copy(idx_hbm.at[pl.ds(base, b_per_w)], idx_v)

        def step2(pi, carry):
            off0 = (2 * pi) * _SC_CH
            off1 = off0 + _SC_CH
            cp0 = pltpu.async_copy(
                table_hbm.at[idx_v.at[pl.ds(off0, _SC_CH)]], rows0, semg0)
            cp1 = pltpu.async_copy(
                table_hbm.at[idx_v.at[pl.ds(off1, _SC_CH)]], rows1, semg1)
            cp0.wait()
            st0 = pltpu.async_copy(
                rows0, out_hbm.at[pl.ds(base + off0, _SC_CH)], semo0)
            cp1.wait()
            st1 = pltpu.async_copy(
                rows1, out_hbm.at[pl.ds(base + off1, _SC_CH)], semo1)
            st0.wait()
            st1.wait()
            return carry

        jax.lax.fori_loop(0, steps // 2, step2, 0)

    return gath(table, idx_flat)


def kernel(features, coords, global_W1, global_b1, global_W2, global_b2,
           local_W1, local_b1, local_W2, local_b2,
           detail_W1, detail_b1, detail_W2, detail_b2,
           prompt_W1, prompt_b1, prompt_W2, prompt_b2):
    B, N, h = features.shape
    hh = h // 2

    W1cat = jnp.concatenate([global_W1, local_W1, detail_W1], axis=1)
    b1cat = jnp.concatenate([global_b1, local_b1, detail_b1], axis=0)
    W2blk = jnp.zeros((3 * hh, 8), jnp.float32)
    W2blk = W2blk.at[0 * hh:1 * hh, 0].set(global_W2[:, 0])
    W2blk = W2blk.at[1 * hh:2 * hh, 1].set(local_W2[:, 0])
    W2blk = W2blk.at[2 * hh:3 * hh, 2].set(detail_W2[:, 0])
    b2cat = jnp.zeros((8,), jnp.float32)
    b2cat = b2cat.at[0].set(global_b2[0]).at[1].set(local_b2[0]).at[2].set(detail_b2[0])

    k_global, k_local, k_detail = N // 8, N // 4, N // 2
    sidx, P = _scores_and_prompt(features, W1cat, b1cat, W2blk, b2cat,
                                 prompt_W1, prompt_b1, prompt_W2, prompt_b2)
    sidx = sidx.reshape(B, 3, N)
    gi = sidx[:, 0, :k_global]
    li = sidx[:, 1, :k_local]
    di = sidx[:, 2, :k_detail]

    idx_all = jnp.concatenate([gi, li, di], axis=1)
    M = idx_all.shape[1]
    out = _sc_gather(P.reshape(B * N, h), idx_all.reshape(B * M))
    return out.reshape(B, M, h)


# speedup vs baseline: 1.0045x; 1.0045x over previous
import functools
import jax
import jax.numpy as jnp
from jax.experimental import pallas as pl
from jax.experimental.pallas import tpu as pltpu
from jax.experimental.pallas import tpu_sc as plsc


def _f32(x):
    return jnp.float32(x)


def _erfc_cephes(x):
    abs_x = jnp.abs(x)
    xx = x * x
    ep = _f32(7.853861353153693e-5)
    for c in (-8.010193625184903e-4, 5.188327685732524e-3,
              -2.685381193529856e-2, 1.128358514861418e-1,
              -3.761262582423300e-1, 1.128379165726710e+0):
        ep = ep * xx + _f32(c)
    branch_lt1 = _f32(1.0) - x * ep
    nxx = -xx
    z = jnp.exp(nxx)
    q = _f32(1.0) / abs_x
    zq = z * q
    w = _f32(1.0) / xx
    pp = _f32(2.326819970068386e-2)
    for c in (-1.387039388740657e-1, 3.687424674597105e-1,
              -5.824733027278666e-1, 6.210004621745983e-1,
              -4.944515323274145e-1, 3.404879937665872e-1,
              -2.741127028184656e-1, 5.638259427386472e-1):
        pp = pp * w + _f32(c)
    rr = _f32(-1.047766399936249e+1)
    for c in (1.297719955372516e+1, -7.495518717768503e+0,
              2.921019019210786e+0, -1.015265279202700e+0,
              4.218463358204948e-1, -2.820767439740514e-1,
              5.641895067754075e-1):
        rr = rr * w + _f32(c)
    p = jnp.where(abs_x < _f32(2.0), pp, rr)
    y = zq * p
    y = jnp.where(nxx < _f32(-88.72283905206835), _f32(0.0), y)
    res_big = jnp.where(x < _f32(0.0), _f32(2.0) - y, y)
    return jnp.where(abs_x < _f32(1.0), branch_lt1, res_big)


_SQRT_HALF = 0.5 ** 0.5


def _gelu(x):
    return 0.5 * x * _erfc_cephes(-x * _f32(_SQRT_HALF))


def _gelu_fast(x):
    return 0.5 * x * (1.0 + jax.lax.erf(x * _f32(_SQRT_HALF)))


def _mlp_body(f_ref, w1_ref, b1_ref, w2_ref, b2_ref, pw1_ref, pb1_ref,
              pw2_ref, pb2_ref, scores_ref, p_ref):
    f = f_ref[0]
    hid = _gelu(jnp.dot(f, w1_ref[...], preferred_element_type=jnp.float32)
                + b1_ref[...])
    sc = jnp.dot(hid, w2_ref[...],
                 preferred_element_type=jnp.float32) + b2_ref[...]
    sct = jnp.transpose(sc, (1, 0))[:3]
    scores_ref[0] = sct.reshape(3, sct.shape[1] // _C, _C)
    ph = _gelu_fast(jnp.dot(f, pw1_ref[...], preferred_element_type=jnp.float32)
                    + pb1_ref[...])
    p_ref[0] = jnp.dot(ph, pw2_ref[...],
                       preferred_element_type=jnp.float32) + pb2_ref[...]


def _scores_and_prompt(features, W1cat, b1cat, W2blk, b2cat,
                       prompt_W1, prompt_b1, prompt_W2, prompt_b2, TN=2048):
    B, N, h = features.shape
    grid = (B, N // TN)
    return pl.pallas_call(
        _mlp_body,
        grid=grid,
        in_specs=[
            pl.BlockSpec((1, TN, h), lambda b, n: (b, n, 0)),
            pl.BlockSpec((h, 384), lambda b, n: (0, 0)),
            pl.BlockSpec((384,), lambda b, n: (0,)),
            pl.BlockSpec((384, 8), lambda b, n: (0, 0)),
            pl.BlockSpec((8,), lambda b, n: (0,)),
            pl.BlockSpec((h, h), lambda b, n: (0, 0)),
            pl.BlockSpec((h,), lambda b, n: (0,)),
            pl.BlockSpec((h, h), lambda b, n: (0, 0)),
            pl.BlockSpec((h,), lambda b, n: (0,)),
        ],
        out_specs=[
            pl.BlockSpec((1, 3, TN // _C, _C), lambda b, n: (b, 0, n, 0)),
            pl.BlockSpec((1, TN, h), lambda b, n: (b, n, 0)),
        ],
        out_shape=[
            jax.ShapeDtypeStruct((B, 3, N // _C, _C), jnp.float32),
            jax.ShapeDtypeStruct((B, N, h), jnp.float32),
        ],
    )(features, W1cat, b1cat, W2blk, b2cat,
      prompt_W1, prompt_b1, prompt_W2, prompt_b2)


_R, _C = 128, 128
_SORT_WAYS = 2


def _partner(x, j):
    if j < _C:
        fwd = pltpu.roll(x, _C - j, axis=1)
        bwd = pltpu.roll(x, j, axis=1)
        bit = (jax.lax.broadcasted_iota(jnp.int32, (_R, _C), 1) & j) == 0
    else:
        m = j // _C
        fwd = pltpu.roll(x, _R - m, axis=0)
        bwd = pltpu.roll(x, m, axis=0)
        bit = (jax.lax.broadcasted_iota(jnp.int32, (_R, _C), 0) & m) == 0
    return jnp.where(bit, fwd, bwd)


def _sort_body(s_ref, idx_ref):
    T = _SORT_WAYS
    r_iota = jax.lax.broadcasted_iota(jnp.int32, (_R, _C), 0)
    c_iota = jax.lax.broadcasted_iota(jnp.int32, (_R, _C), 1)
    pos = r_iota * _C + c_iota
    ss = [s_ref[t] for t in range(T)]
    ii = [pos for _ in range(T)]
    n = _R * _C
    k = 2
    while k <= n:
        j = k // 2
        while j >= 1:
            is_lower = (pos & j) == 0
            block_fwd = (pos & k) == 0
            fwd_dir = block_fwd == is_lower
            for t in range(T):
                ps = _partner(ss[t], j)
                pi = _partner(ii[t], j)
                self_first = (ss[t] > ps) | ((ss[t] == ps) & (ii[t] < pi))
                keep_self = self_first == fwd_dir
                ss[t] = jnp.where(keep_self, ss[t], ps)
                ii[t] = jnp.where(keep_self, ii[t], pi)
            j //= 2
        k *= 2
    for t in range(T):
        base = ((pl.program_id(0) * T + t) // 3) * n
        idx_ref[t] = ii[t] + base


def _argsort_desc(scores):
    G = scores.shape[0]
    T = _SORT_WAYS
    return pl.pallas_call(
        _sort_body,
        grid=(G // T,),
        in_specs=[pl.BlockSpec((T, _R, _C), lambda g: (g, 0, 0))],
        out_specs=pl.BlockSpec((T, _R, _C), lambda g: (g, 0, 0)),
        out_shape=jax.ShapeDtypeStruct((G, _R, _C), jnp.int32),
    )(scores)


_SC_CH = 224


def _sc_gather(table, idx_flat):
    M = idx_flat.shape[0]
    D = table.shape[1]
    info = plsc.get_sparse_core_info()
    NC, NS = info.num_cores, info.num_subcores
    NW = NC * NS
    b_per_w = M // NW
    steps = b_per_w // _SC_CH
    mesh = plsc.VectorSubcoreMesh(core_axis_name="c", subcore_axis_name="s")

    @functools.partial(
        pl.kernel, mesh=mesh,
        out_type=jax.ShapeDtypeStruct((M, D), jnp.float32),
        scratch_types=[
            pltpu.VMEM((b_per_w,), jnp.int32),
            pltpu.VMEM((_SC_CH, D), jnp.float32),
            pltpu.VMEM((_SC_CH, D), jnp.float32),
            pltpu.SemaphoreType.DMA,
            pltpu.SemaphoreType.DMA,
            pltpu.SemaphoreType.DMA,
            pltpu.SemaphoreType.DMA,
        ],
    )
    def gath(table_hbm, idx_hbm, out_hbm, idx_v, rows0, rows1,
             semg0, semg1, semo0, semo1):
        wid = jax.lax.axis_index("s") * NC + jax.lax.axis_index("c")
        base = wid * b_per_w
        pltpu.sync_copy(idx_hbm.at[pl.ds(base, b_per_w)], idx_v)

        def step2(pi, carry):
            off0 = (2 * pi) * _SC_CH
            off1 = off0 + _SC_CH
            cp0 = pltpu.async_copy(
                table_hbm.at[idx_v.at[pl.ds(off0, _SC_CH)]], rows0, semg0)
            cp1 = pltpu.async_copy(
                table_hbm.at[idx_v.at[pl.ds(off1, _SC_CH)]], rows1, semg1)
            cp0.wait()
            st0 = pltpu.async_copy(
                rows0, out_hbm.at[pl.ds(base + off0, _SC_CH)], semo0)
            cp1.wait()
            st1 = pltpu.async_copy(
                rows1, out_hbm.at[pl.ds(base + off1, _SC_CH)], semo1)
            st0.wait()
            st1.wait()
            return carry

        jax.lax.fori_loop(0, steps // 2, step2, 0)

    return gath(table, idx_flat)


def kernel(features, coords, global_W1, global_b1, global_W2, global_b2,
           local_W1, local_b1, local_W2, local_b2,
           detail_W1, detail_b1, detail_W2, detail_b2,
           prompt_W1, prompt_b1, prompt_W2, prompt_b2):
    B, N, h = features.shape
    hh = h // 2

    W1cat = jnp.concatenate([global_W1, local_W1, detail_W1], axis=1)
    b1cat = jnp.concatenate([global_b1, local_b1, detail_b1], axis=0)
    W2blk = jnp.zeros((3 * hh, 8), jnp.float32)
    W2blk = W2blk.at[0 * hh:1 * hh, 0].set(global_W2[:, 0])
    W2blk = W2blk.at[1 * hh:2 * hh, 1].set(local_W2[:, 0])
    W2blk = W2blk.at[2 * hh:3 * hh, 2].set(detail_W2[:, 0])
    b2cat = jnp.zeros((8,), jnp.float32)
    b2cat = b2cat.at[0].set(global_b2[0]).at[1].set(local_b2[0]).at[2].set(detail_b2[0])

    scores, P = _scores_and_prompt(features, W1cat, b1cat, W2blk, b2cat,
                                   prompt_W1, prompt_b1, prompt_W2, prompt_b2)

    k_global, k_local, k_detail = N // 8, N // 4, N // 2
    sidx = _argsort_desc(scores.reshape(3 * B, _R, _C))
    sidx = sidx.reshape(B, 3, N)
    gi = sidx[:, 0, :k_global]
    li = sidx[:, 1, :k_local]
    di = sidx[:, 2, :k_detail]

    idx_all = jnp.concatenate([gi, li, di], axis=1)
    M = idx_all.shape[1]
    out = _sc_gather(P.reshape(B * N, h), idx_all.reshape(B * M))
    return out.reshape(B, M, h)


# speedup vs baseline: 1.0096x; 1.0050x over previous
import functools
import jax
import jax.numpy as jnp
from jax.experimental import pallas as pl
from jax.experimental.pallas import tpu as pltpu
from jax.experimental.pallas import tpu_sc as plsc


def _f32(x):
    return jnp.float32(x)


def _erfc_cephes(x):
    abs_x = jnp.abs(x)
    xx = x * x
    ep = _f32(7.853861353153693e-5)
    for c in (-8.010193625184903e-4, 5.188327685732524e-3,
              -2.685381193529856e-2, 1.128358514861418e-1,
              -3.761262582423300e-1, 1.128379165726710e+0):
        ep = ep * xx + _f32(c)
    branch_lt1 = _f32(1.0) - x * ep
    nxx = -xx
    z = jnp.exp(nxx)
    q = _f32(1.0) / abs_x
    zq = z * q
    w = _f32(1.0) / xx
    pp = _f32(2.326819970068386e-2)
    for c in (-1.387039388740657e-1, 3.687424674597105e-1,
              -5.824733027278666e-1, 6.210004621745983e-1,
              -4.944515323274145e-1, 3.404879937665872e-1,
              -2.741127028184656e-1, 5.638259427386472e-1):
        pp = pp * w + _f32(c)
    rr = _f32(-1.047766399936249e+1)
    for c in (1.297719955372516e+1, -7.495518717768503e+0,
              2.921019019210786e+0, -1.015265279202700e+0,
              4.218463358204948e-1, -2.820767439740514e-1,
              5.641895067754075e-1):
        rr = rr * w + _f32(c)
    p = jnp.where(abs_x < _f32(2.0), pp, rr)
    y = zq * p
    y = jnp.where(nxx < _f32(-88.72283905206835), _f32(0.0), y)
    res_big = jnp.where(x < _f32(0.0), _f32(2.0) - y, y)
    return jnp.where(abs_x < _f32(1.0), branch_lt1, res_big)


_SQRT_HALF = 0.5 ** 0.5


def _gelu(x):
    return 0.5 * x * _erfc_cephes(-x * _f32(_SQRT_HALF))


def _gelu_fast(x):
    return 0.5 * x * (1.0 + jax.lax.erf(x * _f32(_SQRT_HALF)))


def _mlp_body(f_ref, gw1_ref, lw1_ref, dw1_ref, gb1_ref, lb1_ref, db1_ref,
              gw2_ref, lw2_ref, dw2_ref, gb2_ref, lb2_ref, db2_ref,
              pw1_ref, pb1_ref, pw2_ref, pb2_ref, scores_ref, p_ref):
    f = f_ref[0]
    hh = gw1_ref.shape[1]
    w1 = jnp.concatenate([gw1_ref[...], lw1_ref[...], dw1_ref[...]], axis=1)
    b1 = jnp.concatenate([gb1_ref[...], lb1_ref[...], db1_ref[...]], axis=0)
    z8 = jnp.zeros((hh, 1), jnp.float32)
    w2 = jnp.concatenate([
        jnp.concatenate([gw2_ref[...], z8, z8, z8, z8, z8, z8, z8], axis=1),
        jnp.concatenate([z8, lw2_ref[...], z8, z8, z8, z8, z8, z8], axis=1),
        jnp.concatenate([z8, z8, dw2_ref[...], z8, z8, z8, z8, z8], axis=1),
    ], axis=0)
    b2 = jnp.concatenate([gb2_ref[...], lb2_ref[...], db2_ref[...],
                          jnp.zeros((5,), jnp.float32)], axis=0)
    hid = _gelu(jnp.dot(f, w1, preferred_element_type=jnp.float32) + b1)
    sc = jnp.dot(hid, w2, preferred_element_type=jnp.float32) + b2
    sct = jnp.transpose(sc, (1, 0))[:3]
    scores_ref[0] = sct.reshape(3, sct.shape[1] // _C, _C)
    ph = _gelu_fast(jnp.dot(f, pw1_ref[...], preferred_element_type=jnp.float32)
                    + pb1_ref[...])
    p_ref[0] = jnp.dot(ph, pw2_ref[...],
                       preferred_element_type=jnp.float32) + pb2_ref[...]


def _scores_and_prompt(features, gW1, gb1, gW2, gb2, lW1, lb1, lW2, lb2,
                       dW1, db1, dW2, db2,
                       prompt_W1, prompt_b1, prompt_W2, prompt_b2, TN=2048):
    B, N, h = features.shape
    hh = h // 2
    grid = (B, N // TN)
    full = lambda b, n: (0, 0)
    full1 = lambda b, n: (0,)
    return pl.pallas_call(
        _mlp_body,
        grid=grid,
        in_specs=[
            pl.BlockSpec((1, TN, h), lambda b, n: (b, n, 0)),
            pl.BlockSpec((h, hh), full), pl.BlockSpec((h, hh), full),
            pl.BlockSpec((h, hh), full),
            pl.BlockSpec((hh,), full1), pl.BlockSpec((hh,), full1),
            pl.BlockSpec((hh,), full1),
            pl.BlockSpec((hh, 1), full), pl.BlockSpec((hh, 1), full),
            pl.BlockSpec((hh, 1), full),
            pl.BlockSpec((1,), full1), pl.BlockSpec((1,), full1),
            pl.BlockSpec((1,), full1),
            pl.BlockSpec((h, h), full),
            pl.BlockSpec((h,), full1),
            pl.BlockSpec((h, h), full),
            pl.BlockSpec((h,), full1),
        ],
        out_specs=[
            pl.BlockSpec((1, 3, TN // _C, _C), lambda b, n: (b, 0, n, 0)),
            pl.BlockSpec((1, TN, h), lambda b, n: (b, n, 0)),
        ],
        out_shape=[
            jax.ShapeDtypeStruct((B, 3, N // _C, _C), jnp.float32),
            jax.ShapeDtypeStruct((B, N, h), jnp.float32),
        ],
    )(features, gW1, lW1, dW1, gb1, lb1, db1, gW2, lW2, dW2, gb2, lb2, db2,
      prompt_W1, prompt_b1, prompt_W2, prompt_b2)


_R, _C = 128, 128
_SORT_WAYS = 2


def _partner(x, j):
    if j < _C:
        fwd = pltpu.roll(x, _C - j, axis=1)
        bwd = pltpu.roll(x, j, axis=1)
        bit = (jax.lax.broadcasted_iota(jnp.int32, (_R, _C), 1) & j) == 0
    else:
        m = j // _C
        fwd = pltpu.roll(x, _R - m, axis=0)
        bwd = pltpu.roll(x, m, axis=0)
        bit = (jax.lax.broadcasted_iota(jnp.int32, (_R, _C), 0) & m) == 0
    return jnp.where(bit, fwd, bwd)


def _sort_body(s_ref, idx_ref):
    T = _SORT_WAYS
    r_iota = jax.lax.broadcasted_iota(jnp.int32, (_R, _C), 0)
    c_iota = jax.lax.broadcasted_iota(jnp.int32, (_R, _C), 1)
    pos = r_iota * _C + c_iota
    ss = [s_ref[t] for t in range(T)]
    ii = [pos for _ in range(T)]
    n = _R * _C
    k = 2
    while k <= n:
        j = k // 2
        while j >= 1:
            is_lower = (pos & j) == 0
            block_fwd = (pos & k) == 0
            fwd_dir = block_fwd == is_lower
            for t in range(T):
                ps = _partner(ss[t], j)
                pi = _partner(ii[t], j)
                self_first = (ss[t] > ps) | ((ss[t] == ps) & (ii[t] < pi))
                keep_self = self_first == fwd_dir
                ss[t] = jnp.where(keep_self, ss[t], ps)
                ii[t] = jnp.where(keep_self, ii[t], pi)
            j //= 2
        k *= 2
    for t in range(T):
        base = ((pl.program_id(0) * T + t) // 3) * n
        idx_ref[t] = ii[t] + base


def _argsort_desc(scores):
    G = scores.shape[0]
    T = _SORT_WAYS
    return pl.pallas_call(
        _sort_body,
        grid=(G // T,),
        in_specs=[pl.BlockSpec((T, _R, _C), lambda g: (g, 0, 0))],
        out_specs=pl.BlockSpec((T, _R, _C), lambda g: (g, 0, 0)),
        out_shape=jax.ShapeDtypeStruct((G, _R, _C), jnp.int32),
    )(scores)


_SC_CH = 112
_SC_NBUF = 4


def _sc_gather(table, idx_flat):
    M = idx_flat.shape[0]
    D = table.shape[1]
    info = plsc.get_sparse_core_info()
    NC, NS = info.num_cores, info.num_subcores
    NW = NC * NS
    b_per_w = M // NW
    steps = b_per_w // _SC_CH
    mesh = plsc.VectorSubcoreMesh(core_axis_name="c", subcore_axis_name="s")

    @functools.partial(
        pl.kernel, mesh=mesh,
        out_type=jax.ShapeDtypeStruct((M, D), jnp.float32),
        scratch_types=(
            [pltpu.VMEM((b_per_w,), jnp.int32)]
            + [pltpu.VMEM((_SC_CH, D), jnp.float32)] * _SC_NBUF
            + [pltpu.SemaphoreType.DMA] * (2 * _SC_NBUF)
        ),
    )
    def gath(table_hbm, idx_hbm, out_hbm, idx_v, *bufs_and_sems):
        rows = bufs_and_sems[:_SC_NBUF]
        semg = bufs_and_sems[_SC_NBUF:2 * _SC_NBUF]
        semo = bufs_and_sems[2 * _SC_NBUF:]
        wid = jax.lax.axis_index("s") * NC + jax.lax.axis_index("c")
        base = wid * b_per_w
        pltpu.sync_copy(idx_hbm.at[pl.ds(base, b_per_w)], idx_v)

        def stepn(pi, carry):
            offs = [(pi * _SC_NBUF + q) * _SC_CH for q in range(_SC_NBUF)]
            gets = [pltpu.async_copy(
                table_hbm.at[idx_v.at[pl.ds(offs[q], _SC_CH)]],
                rows[q], semg[q]) for q in range(_SC_NBUF)]
            puts = []
            for q in range(_SC_NBUF):
                gets[q].wait()
                puts.append(pltpu.async_copy(
                    rows[q], out_hbm.at[pl.ds(base + offs[q], _SC_CH)],
                    semo[q]))
            for q in range(_SC_NBUF):
                puts[q].wait()
            return carry

        jax.lax.fori_loop(0, steps // _SC_NBUF, stepn, 0)

    return gath(table, idx_flat)


def kernel(features, coords, global_W1, global_b1, global_W2, global_b2,
           local_W1, local_b1, local_W2, local_b2,
           detail_W1, detail_b1, detail_W2, detail_b2,
           prompt_W1, prompt_b1, prompt_W2, prompt_b2):
    B, N, h = features.shape

    scores, P = _scores_and_prompt(
        features, global_W1, global_b1, global_W2, global_b2,
        local_W1, local_b1, local_W2, local_b2,
        detail_W1, detail_b1, detail_W2, detail_b2,
        prompt_W1, prompt_b1, prompt_W2, prompt_b2)

    k_global, k_local, k_detail = N // 8, N // 4, N // 2
    sidx = _argsort_desc(scores.reshape(3 * B, _R, _C))
    sidx = sidx.reshape(B, 3, N)
    gi = sidx[:, 0, :k_global]
    li = sidx[:, 1, :k_local]
    di = sidx[:, 2, :k_detail]

    idx_all = jnp.concatenate([gi, li, di], axis=1)
    M = idx_all.shape[1]
    out = _sc_gather(P.reshape(B * N, h), idx_all.reshape(B * M))
    return out.reshape(B, M, h)


# speedup vs baseline: 1.0396x; 1.0297x over previous
import functools
import jax
import jax.numpy as jnp
from jax.experimental import pallas as pl
from jax.experimental.pallas import tpu as pltpu
from jax.experimental.pallas import tpu_sc as plsc


def _f32(x):
    return jnp.float32(x)


def _erfc_cephes(x):
    abs_x = jnp.abs(x)
    xx = x * x
    ep = _f32(7.853861353153693e-5)
    for c in (-8.010193625184903e-4, 5.188327685732524e-3,
              -2.685381193529856e-2, 1.128358514861418e-1,
              -3.761262582423300e-1, 1.128379165726710e+0):
        ep = ep * xx + _f32(c)
    branch_lt1 = _f32(1.0) - x * ep
    nxx = -xx
    z = jnp.exp(nxx)
    q = _f32(1.0) / abs_x
    zq = z * q
    w = _f32(1.0) / xx
    pp = _f32(2.326819970068386e-2)
    for c in (-1.387039388740657e-1, 3.687424674597105e-1,
              -5.824733027278666e-1, 6.210004621745983e-1,
              -4.944515323274145e-1, 3.404879937665872e-1,
              -2.741127028184656e-1, 5.638259427386472e-1):
        pp = pp * w + _f32(c)
    rr = _f32(-1.047766399936249e+1)
    for c in (1.297719955372516e+1, -7.495518717768503e+0,
              2.921019019210786e+0, -1.015265279202700e+0,
              4.218463358204948e-1, -2.820767439740514e-1,
              5.641895067754075e-1):
        rr = rr * w + _f32(c)
    p = jnp.where(abs_x < _f32(2.0), pp, rr)
    y = zq * p
    y = jnp.where(nxx < _f32(-88.72283905206835), _f32(0.0), y)
    res_big = jnp.where(x < _f32(0.0), _f32(2.0) - y, y)
    return jnp.where(abs_x < _f32(1.0), branch_lt1, res_big)


_SQRT_HALF = 0.5 ** 0.5


def _gelu(x):
    return 0.5 * x * _erfc_cephes(-x * _f32(_SQRT_HALF))


def _gelu_fast(x):
    return 0.5 * x * (1.0 + jax.lax.erf(x * _f32(_SQRT_HALF)))


def _mlp_body(f_ref, gw1_ref, lw1_ref, dw1_ref, gb1_ref, lb1_ref, db1_ref,
              gw2_ref, lw2_ref, dw2_ref, gb2_ref, lb2_ref, db2_ref,
              pw1_ref, pb1_ref, pw2_ref, pb2_ref, scores_ref, p_ref):
    f = f_ref[0]
    hh = gw1_ref.shape[1]
    w1 = jnp.concatenate([gw1_ref[...], lw1_ref[...], dw1_ref[...]], axis=1)
    b1 = jnp.concatenate([gb1_ref[...], lb1_ref[...], db1_ref[...]], axis=0)
    z8 = jnp.zeros((hh, 1), jnp.float32)
    w2 = jnp.concatenate([
        jnp.concatenate([gw2_ref[...], z8, z8, z8, z8, z8, z8, z8], axis=1),
        jnp.concatenate([z8, lw2_ref[...], z8, z8, z8, z8, z8, z8], axis=1),
        jnp.concatenate([z8, z8, dw2_ref[...], z8, z8, z8, z8, z8], axis=1),
    ], axis=0)
    b2 = jnp.concatenate([gb2_ref[...], lb2_ref[...], db2_ref[...],
                          jnp.zeros((5,), jnp.float32)], axis=0)
    hid = _gelu(jnp.dot(f, w1, preferred_element_type=jnp.float32) + b1)
    sc = jnp.dot(hid, w2, preferred_element_type=jnp.float32) + b2
    sct = jnp.transpose(sc, (1, 0))[:3]
    scores_ref[0] = sct.reshape(3, sct.shape[1] // _C, _C)
    ph = _gelu_fast(jnp.dot(f, pw1_ref[...], preferred_element_type=jnp.float32)
                    + pb1_ref[...])
    p_ref[0] = jnp.dot(ph, pw2_ref[...],
                       preferred_element_type=jnp.float32) + pb2_ref[...]


def _scores_and_prompt(features, gW1, gb1, gW2, gb2, lW1, lb1, lW2, lb2,
                       dW1, db1, dW2, db2,
                       prompt_W1, prompt_b1, prompt_W2, prompt_b2, TN=4096):
    B, N, h = features.shape
    hh = h // 2
    grid = (B, N // TN)
    full = lambda b, n: (0, 0)
    full1 = lambda b, n: (0,)
    return pl.pallas_call(
        _mlp_body,
        grid=grid,
        in_specs=[
            pl.BlockSpec((1, TN, h), lambda b, n: (b, n, 0)),
            pl.BlockSpec((h, hh), full), pl.BlockSpec((h, hh), full),
            pl.BlockSpec((h, hh), full),
            pl.BlockSpec((hh,), full1), pl.BlockSpec((hh,), full1),
            pl.BlockSpec((hh,), full1),
            pl.BlockSpec((hh, 1), full), pl.BlockSpec((hh, 1), full),
            pl.BlockSpec((hh, 1), full),
            pl.BlockSpec((1,), full1), pl.BlockSpec((1,), full1),
            pl.BlockSpec((1,), full1),
            pl.BlockSpec((h, h), full),
            pl.BlockSpec((h,), full1),
            pl.BlockSpec((h, h), full),
            pl.BlockSpec((h,), full1),
        ],
        out_specs=[
            pl.BlockSpec((1, 3, TN // _C, _C), lambda b, n: (b, 0, n, 0)),
            pl.BlockSpec((1, TN, h), lambda b, n: (b, n, 0)),
        ],
        out_shape=[
            jax.ShapeDtypeStruct((B, 3, N // _C, _C), jnp.float32),
            jax.ShapeDtypeStruct((B, N, h), jnp.float32),
        ],
    )(features, gW1, lW1, dW1, gb1, lb1, db1, gW2, lW2, dW2, gb2, lb2, db2,
      prompt_W1, prompt_b1, prompt_W2, prompt_b2)


_R, _C = 128, 128
_SORT_WAYS = 2


def _partner(x, j):
    if j < _C:
        fwd = pltpu.roll(x, _C - j, axis=1)
        bwd = pltpu.roll(x, j, axis=1)
        bit = (jax.lax.broadcasted_iota(jnp.int32, (_R, _C), 1) & j) == 0
    else:
        m = j // _C
        fwd = pltpu.roll(x, _R - m, axis=0)
        bwd = pltpu.roll(x, m, axis=0)
        bit = (jax.lax.broadcasted_iota(jnp.int32, (_R, _C), 0) & m) == 0
    return jnp.where(bit, fwd, bwd)


def _sort_body(s_ref, idx_ref):
    T = _SORT_WAYS
    r_iota = jax.lax.broadcasted_iota(jnp.int32, (_R, _C), 0)
    c_iota = jax.lax.broadcasted_iota(jnp.int32, (_R, _C), 1)
    pos = r_iota * _C + c_iota
    ss = [s_ref[t] for t in range(T)]
    ii = [pos for _ in range(T)]
    n = _R * _C
    k = 2
    while k <= n:
        j = k // 2
        while j >= 1:
            is_lower = (pos & j) == 0
            block_fwd = (pos & k) == 0
            fwd_dir = block_fwd == is_lower
            for t in range(T):
                ps = _partner(ss[t], j)
                pi = _partner(ii[t], j)
                self_first = (ss[t] > ps) | ((ss[t] == ps) & (ii[t] < pi))
                keep_self = self_first == fwd_dir
                ss[t] = jnp.where(keep_self, ss[t], ps)
                ii[t] = jnp.where(keep_self, ii[t], pi)
            j //= 2
        k *= 2
    for t in range(T):
        base = ((pl.program_id(0) * T + t) // 3) * n
        idx_ref[t] = ii[t] + base


def _argsort_desc(scores):
    G = scores.shape[0]
    T = _SORT_WAYS
    return pl.pallas_call(
        _sort_body,
        grid=(G // T,),
        in_specs=[pl.BlockSpec((T, _R, _C), lambda g: (g, 0, 0))],
        out_specs=pl.BlockSpec((T, _R, _C), lambda g: (g, 0, 0)),
        out_shape=jax.ShapeDtypeStruct((G, _R, _C), jnp.int32),
    )(scores)


_SC_CH = 112
_SC_NBUF = 4


def _sc_gather(table, idx_flat):
    M = idx_flat.shape[0]
    D = table.shape[1]
    info = plsc.get_sparse_core_info()
    NC, NS = info.num_cores, info.num_subcores
    NW = NC * NS
    b_per_w = M // NW
    steps = b_per_w // _SC_CH
    mesh = plsc.VectorSubcoreMesh(core_axis_name="c", subcore_axis_name="s")

    @functools.partial(
        pl.kernel, mesh=mesh,
        out_type=jax.ShapeDtypeStruct((M, D), jnp.float32),
        scratch_types=(
            [pltpu.VMEM((b_per_w,), jnp.int32)]
            + [pltpu.VMEM((_SC_CH, D), jnp.float32)] * _SC_NBUF
            + [pltpu.SemaphoreType.DMA] * (2 * _SC_NBUF)
        ),
    )
    def gath(table_hbm, idx_hbm, out_hbm, idx_v, *bufs_and_sems):
        rows = bufs_and_sems[:_SC_NBUF]
        semg = bufs_and_sems[_SC_NBUF:2 * _SC_NBUF]
        semo = bufs_and_sems[2 * _SC_NBUF:]
        wid = jax.lax.axis_index("s") * NC + jax.lax.axis_index("c")
        base = wid * b_per_w
        pltpu.sync_copy(idx_hbm.at[pl.ds(base, b_per_w)], idx_v)

        def stepn(pi, carry):
            offs = [(pi * _SC_NBUF + q) * _SC_CH for q in range(_SC_NBUF)]
            gets = [pltpu.async_copy(
                table_hbm.at[idx_v.at[pl.ds(offs[q], _SC_CH)]],
                rows[q], semg[q]) for q in range(_SC_NBUF)]
            puts = []
            for q in range(_SC_NBUF):
                gets[q].wait()
                puts.append(pltpu.async_copy(
                    rows[q], out_hbm.at[pl.ds(base + offs[q], _SC_CH)],
                    semo[q]))
            for q in range(_SC_NBUF):
                puts[q].wait()
            return carry

        jax.lax.fori_loop(0, steps // _SC_NBUF, stepn, 0)

    return gath(table, idx_flat)


def kernel(features, coords, global_W1, global_b1, global_W2, global_b2,
           local_W1, local_b1, local_W2, local_b2,
           detail_W1, detail_b1, detail_W2, detail_b2,
           prompt_W1, prompt_b1, prompt_W2, prompt_b2):
    B, N, h = features.shape

    scores, P = _scores_and_prompt(
        features, global_W1, global_b1, global_W2, global_b2,
        local_W1, local_b1, local_W2, local_b2,
        detail_W1, detail_b1, detail_W2, detail_b2,
        prompt_W1, prompt_b1, prompt_W2, prompt_b2)

    k_global, k_local, k_detail = N // 8, N // 4, N // 2
    sidx = _argsort_desc(scores.reshape(3 * B, _R, _C))
    sidx = sidx.reshape(B, 3, N)
    gi = sidx[:, 0, :k_global]
    li = sidx[:, 1, :k_local]
    di = sidx[:, 2, :k_detail]

    idx_all = jnp.concatenate([gi, li, di], axis=1)
    M = idx_all.shape[1]
    out = _sc_gather(P.reshape(B * N, h), idx_all.reshape(B * M))
    return out.reshape(B, M, h)
